# baseline (speedup 1.0000x reference)
import functools

import jax
import jax.numpy as jnp
from jax import lax
from jax.experimental import pallas as pl
from jax.experimental.pallas import tpu as pltpu
from jax.experimental.pallas import tpu_sc as plsc

NC = 2
NS = 16
NW = NC * NS
B = 4096
D = 64
L = 50
MARGIN = 1.0
RPW = B // NW
KG = D // 16
NBUF = 2


def _tcf_body(uid_h, pid_h, nid_h, unbr_h, pnbr_h, nnbr_h, utab_h, itab_h,
              out_h,
              uidx_v, pidx_v, nidx_v, urows_v, prows_v, nrows_v,
              uni_v, pni_v, nni_v, ubag_v, pbag_v, nbag_v, out_v,
              ssem, bsem):
    wid = lax.axis_index("s") * NC + lax.axis_index("c")
    base = wid * RPW

    pltpu.sync_copy(uid_h.at[pl.ds(base, RPW)], uidx_v)
    pltpu.sync_copy(pid_h.at[pl.ds(base, RPW)], pidx_v)
    pltpu.sync_copy(nid_h.at[pl.ds(base, RPW)], nidx_v)
    cu = pltpu.async_copy(utab_h.at[uidx_v], urows_v, ssem)
    cp = pltpu.async_copy(itab_h.at[pidx_v], prows_v, ssem)
    cn = pltpu.async_copy(itab_h.at[nidx_v], nrows_v, ssem)
    pltpu.sync_copy(unbr_h.at[pl.ds(base, RPW)], uni_v)
    pltpu.sync_copy(pnbr_h.at[pl.ds(base, RPW)], pni_v)
    pltpu.sync_copy(nnbr_h.at[pl.ds(base, RPW)], nni_v)

    def start_row(i):
        slot = lax.rem(i, NBUF)
        pltpu.async_copy(itab_h.at[uni_v.at[i]], ubag_v.at[slot],
                         bsem.at[slot])
        pltpu.async_copy(utab_h.at[pni_v.at[i]], pbag_v.at[slot],
                         bsem.at[slot])
        pltpu.async_copy(utab_h.at[nni_v.at[i]], nbag_v.at[slot],
                         bsem.at[slot])

    def wait_row(i):
        slot = lax.rem(i, NBUF)
        pltpu.make_async_copy(itab_h.at[uni_v.at[i]], ubag_v.at[slot],
                              bsem.at[slot]).wait()
        pltpu.make_async_copy(utab_h.at[pni_v.at[i]], pbag_v.at[slot],
                              bsem.at[slot]).wait()
        pltpu.make_async_copy(utab_h.at[nni_v.at[i]], nbag_v.at[slot],
                              bsem.at[slot]).wait()

    for i in range(NBUF - 1):
        start_row(i)
    cu.wait()
    cp.wait()
    cn.wait()

    inv_l = jnp.float32(1.0 / L)
    zero = jnp.zeros((16,), jnp.float32)

    def row_body(i, acc):
        @pl.when(i + (NBUF - 1) < RPW)
        def _():
            start_row(i + (NBUF - 1))

        wait_row(i)
        slot = lax.rem(i, NBUF)

        def red(j, c):
            outs = []
            for t, bag in enumerate((ubag_v, pbag_v, nbag_v)):
                for k in range(KG):
                    outs.append(c[t * KG + k]
                                + bag[slot, j, pl.ds(k * 16, 16)])
            return tuple(outs)

        sums = lax.fori_loop(0, L, red, (zero,) * (3 * KG))
        new = []
        for k in range(KG):
            ub = sums[k] * inv_l
            pb = sums[KG + k] * inv_l
            nb = sums[2 * KG + k] * inv_l
            u = urows_v[i, pl.ds(k * 16, 16)]
            pe = prows_v[i, pl.ds(k * 16, 16)]
            ne = nrows_v[i, pl.ds(k * 16, 16)]
            tpos = u + ub * pb - pe
            tneg = u + ub * nb - ne
            v = MARGIN + tpos * tpos - tneg * tneg
            new.append(acc[k] + jnp.maximum(v, 0.0))
        return tuple(new)

    acc = lax.fori_loop(0, RPW, row_body, (zero,) * KG)
    out_v[0, :] = acc[0] + acc[1] + acc[2] + acc[3]
    pltpu.sync_copy(out_v, out_h.at[pl.ds(wid, 1)])


def kernel(user_ids, pos_ids, neg_ids, user_nbr_items, pos_item_nbr_users,
           neg_item_nbr_users, user_table, item_table):
    uid = user_ids.astype(jnp.int32)
    pid = pos_ids.astype(jnp.int32)
    nid = neg_ids.astype(jnp.int32)
    unbr = user_nbr_items.astype(jnp.int32)
    pnbr = pos_item_nbr_users.astype(jnp.int32)
    nnbr = neg_item_nbr_users.astype(jnp.int32)

    mesh = plsc.VectorSubcoreMesh(core_axis_name="c", subcore_axis_name="s")
    run = pl.kernel(
        _tcf_body,
        mesh=mesh,
        compiler_params=pltpu.CompilerParams(use_tc_tiling_on_sc=False),
        out_type=jax.ShapeDtypeStruct((NW, 16), jnp.float32),
        scratch_types=[
            pltpu.VMEM((RPW,), jnp.int32),
            pltpu.VMEM((RPW,), jnp.int32),
            pltpu.VMEM((RPW,), jnp.int32),
            pltpu.VMEM((RPW, D), jnp.float32),
            pltpu.VMEM((RPW, D), jnp.float32),
            pltpu.VMEM((RPW, D), jnp.float32),
            pltpu.VMEM((RPW, L), jnp.int32),
            pltpu.VMEM((RPW, L), jnp.int32),
            pltpu.VMEM((RPW, L), jnp.int32),
            pltpu.VMEM((NBUF, L, D), jnp.float32),
            pltpu.VMEM((NBUF, L, D), jnp.float32),
            pltpu.VMEM((NBUF, L, D), jnp.float32),
            pltpu.VMEM((1, 16), jnp.float32),
            pltpu.SemaphoreType.DMA,
            pltpu.SemaphoreType.DMA((NBUF,)),
        ],
    )
    partials = run(uid, pid, nid, unbr, pnbr, nnbr, user_table, item_table)
    return jnp.sum(partials)
